# baseline (speedup 1.0000x reference)
import jax
import jax.numpy as jnp
from jax.experimental import pallas as pl
from jax.experimental.pallas import tpu as pltpu

_RES = 0.8
_POINT_Z = 0.8
_NB = 20
_TCHUNK = 1


def _box_mask(bx, H, W):
    c = jnp.cos(bx[:, 6])
    s = jnp.sin(bx[:, 6])
    k1 = c * bx[:, 0] + s * bx[:, 1]
    k2 = -s * bx[:, 0] + c * bx[:, 1]
    adx2 = jnp.abs(bx[:, 3]) * 0.5
    ady2 = jnp.abs(bx[:, 4]) * 0.5
    adz2 = jnp.abs(bx[:, 5]) * 0.5
    zok = jnp.abs(_POINT_Z - bx[:, 2]) <= adz2
    adx2 = jnp.where(zok, adx2, -1.0)

    xs_r = (jax.lax.broadcasted_iota(jnp.int32, (H, 1), 0).astype(jnp.float32)
            - H / 2.0) * _RES
    ys_c = (jax.lax.broadcasted_iota(jnp.int32, (1, W), 1).astype(jnp.float32)
            - W / 2.0) * _RES

    mask = None
    for nb in range(_NB):
        ax = c[nb] * xs_r - k1[nb]
        bxv = s[nb] * ys_c
        ay = -s[nb] * xs_r - k2[nb]
        byv = c[nb] * ys_c
        ins = (jnp.abs(ax + bxv) <= adx2[nb]) \
            & (jnp.abs(ay + byv) <= ady2[nb])
        mask = ins if mask is None else (mask | ins)
    return mask.astype(jnp.float32)


def _tree_sum(arrs):
    while len(arrs) > 1:
        nxt = [arrs[i] + arrs[i + 1] for i in range(0, len(arrs) - 1, 2)]
        if len(arrs) % 2:
            nxt.append(arrs[-1])
        arrs = nxt
    return arrs[0]


def _loss_kernel(boxes_ref, *refs):
    n_a = 20 // _TCHUNK
    a_refs = refs[:n_a]
    o_refs = refs[n_a:n_a + 20]
    out_ref = refs[n_a + 20]
    H, W = a_refs[0].shape[2], a_refs[0].shape[3]

    if _TCHUNK == 1:
        pred = _tree_sum([r[0, 0] for r in a_refs])
    else:
        pred = _tree_sum([jnp.sum(r[0], axis=0) for r in a_refs])
    orig = _tree_sum([r[0, 0] for r in o_refs])

    pred_g = (pred > 0.0).astype(jnp.float32)
    orig_g = (orig > 0.0).astype(jnp.float32)
    maskf = _box_mask(boxes_ref[0], H, W)
    inter = jnp.sum(pred_g * orig_g * maskf, keepdims=True)
    union = jnp.sum(jnp.maximum(pred_g, orig_g) * maskf, keepdims=True)
    iou = inter / (union + 1e-6)
    out_ref[...] = iou[None]


def kernel(added_points, original_points, boxes, tf_ego):
    B, T, H, W = added_points.shape
    boxes_p = jnp.zeros((B, 32, 8), dtype=jnp.float32)
    boxes_p = boxes_p.at[:, : boxes.shape[1], :7].set(boxes)
    n_a = T // _TCHUNK

    def _a_spec(j):
        return pl.BlockSpec((1, _TCHUNK, H, W), lambda b, j=j: (b, j, 0, 0))

    def _o_spec(j):
        return pl.BlockSpec((1, 1, H, W), lambda b, j=j: (b, j + 1, 0, 0))

    out = pl.pallas_call(
        _loss_kernel,
        grid=(B,),
        in_specs=[pl.BlockSpec((1, 32, 8), lambda b: (b, 0, 0))]
        + [_a_spec(j) for j in range(n_a)]
        + [_o_spec(j) for j in range(T)],
        out_specs=pl.BlockSpec((1, 1, 1), lambda b: (b, 0, 0)),
        out_shape=jax.ShapeDtypeStruct((B, 1, 1), jnp.float32),
        compiler_params=pltpu.CompilerParams(
            dimension_semantics=("arbitrary",),
            vmem_limit_bytes=110 * 1024 * 1024,
        ),
    )(boxes_p, *([added_points] * n_a), *([original_points] * T))
    return jnp.sum(out) / B
